# baseline (speedup 1.0000x reference)
import functools

import jax
import jax.numpy as jnp
from jax import lax
from jax.experimental import pallas as pl
from jax.experimental.pallas import tpu as pltpu
from jax.experimental.pallas import tpu_sc as plsc

N_NODES = 10000
N_EDGES = 320000
D = 128
DE = 16

NC = 2
NS = 16
NW = NC * NS
NSPLIT = 2
E_SPLIT = N_EDGES // NSPLIT
E_W = E_SPLIT // NW
CH = 40
NCHUNK = E_W // CH
RPT = 624


def _leaky(v):
    return jnp.maximum(v, 0.01 * v)


def _node_dense(x, Wn, bn, Wgn):
    def body(x_ref, wn_ref, bn_ref, wgn_ref, o_ref):
        h = jnp.dot(x_ref[...], wn_ref[...], preferred_element_type=jnp.float32)
        h = _leaky(h + bn_ref[...])
        o_ref[...] = jnp.dot(h, wgn_ref[...], preferred_element_type=jnp.float32)

    return pl.pallas_call(
        body,
        out_shape=jax.ShapeDtypeStruct((N_NODES, D), jnp.float32),
    )(x, Wn, bn, Wgn)


def _edge_dense(ea_t, We, be, Wge, bg, half):
    BLK = 16000
    assert E_SPLIT % BLK == 0
    grid = E_SPLIT // BLK
    boff = half * grid
    cdim = (((0,), (0,)), ((), ()))

    def body(a_ref, we_ref, be_ref, wge_ref, bg_ref, o_ref):
        ht = lax.dot_general(we_ref[...], a_ref[...], cdim,
                             preferred_element_type=jnp.float32)
        ht = _leaky(ht + be_ref[...])
        o_ref[...] = lax.dot_general(ht, wge_ref[...], cdim,
                                     preferred_element_type=jnp.float32) + bg_ref[...]

    return pl.pallas_call(
        body,
        grid=(grid,),
        in_specs=[
            pl.BlockSpec((DE, BLK), lambda i: (0, i + boff)),
            pl.BlockSpec((DE, DE), lambda i: (0, 0)),
            pl.BlockSpec((DE, 1), lambda i: (0, 0)),
            pl.BlockSpec((DE, D), lambda i: (0, 0)),
            pl.BlockSpec((1, D), lambda i: (0, 0)),
        ],
        out_specs=pl.BlockSpec((BLK, D), lambda i: (i, 0)),
        out_shape=jax.ShapeDtypeStruct((E_SPLIT, D), jnp.float32),
    )(ea_t, We, be, Wge, bg)


NBUF = 4
NIDX = 8
UNROLL = 8


def _make_sc_scatter(half):
  eoff = half * E_SPLIT

  @functools.partial(
      pl.kernel,
      out_type=jax.ShapeDtypeStruct((2 * N_NODES, D), jnp.float32),
      mesh=plsc.VectorSubcoreMesh(core_axis_name="c", subcore_axis_name="s"),
      scratch_types=[
          [pltpu.VMEM((2, CH), jnp.int32) for _ in range(NIDX)],
          [pltpu.VMEM((CH, D), jnp.float32) for _ in range(NBUF)],
          [pltpu.VMEM((CH, D), jnp.float32) for _ in range(NBUF)],
          [pltpu.SemaphoreType.DMA for _ in range(NIDX)],
          [pltpu.SemaphoreType.DMA for _ in range(NBUF)],
          [pltpu.SemaphoreType.DMA for _ in range(NBUF)],
          [pltpu.SemaphoreType.DMA for _ in range(NBUF)],
          pltpu.VMEM_SHARED((N_NODES, D), jnp.float32),
      ],
  )
  def _sc_scatter(node_hbm, edge_hbm, src_hbm, dst_hbm, out_hbm,
                  idxs, nbufs, ebufs, isems, gsems, lsems, ssems, acc):
    c = lax.axis_index("c")
    s = lax.axis_index("s")
    wid = c * NS + s
    ebase = wid * E_W
    gbase = eoff + ebase

    z = ebufs[0]

    @plsc.parallel_loop(0, CH, unroll=4)
    def _(e):
        for j in range(8):
            z[e, pl.ds(16 * j, 16)] = jnp.zeros((16,), jnp.float32)

    row0 = s * RPT
    for k in range(15):
        pltpu.sync_copy(z, acc.at[pl.ds(row0 + k * CH, CH)])
    pltpu.sync_copy(z.at[pl.ds(0, 24)], acc.at[pl.ds(row0 + 600, 24)])

    @pl.when(s == NS - 1)
    def _():
        pltpu.sync_copy(z.at[pl.ds(0, 16)], acc.at[pl.ds(row0 + RPT, 16)])

    plsc.subcore_barrier()

    def issue_idx(i, q):
        pltpu.async_copy(src_hbm.at[pl.ds(gbase + i * CH, CH)], idxs[q].at[0], isems[q])
        pltpu.async_copy(dst_hbm.at[pl.ds(gbase + i * CH, CH)], idxs[q].at[1], isems[q])

    def wait_idx(q):
        pltpu.make_async_copy(src_hbm.at[pl.ds(0, CH)], idxs[q].at[0], isems[q]).wait()
        pltpu.make_async_copy(src_hbm.at[pl.ds(0, CH)], idxs[q].at[1], isems[q]).wait()

    def issue_data(i, b, q):
        pltpu.async_copy(node_hbm.at[idxs[q].at[0]], nbufs[b], gsems[b])
        pltpu.async_copy(edge_hbm.at[pl.ds(ebase + i * CH, CH)], ebufs[b], lsems[b])

    def wait_scatter(b):
        pltpu.make_async_copy(ebufs[b], acc.at[idxs[0].at[1]], ssems[b]).wait()

    for j in range(3):
        issue_idx(j, j)
    for j in range(2):
        wait_idx(j)
        issue_data(j, j, j)

    def step(i, b, q):
        b2 = (b + 2) % NBUF
        q2 = (q + 2) % NIDX
        q3 = (q + 3) % NIDX
        i2 = i + 2
        i3 = i + 3

        @pl.when(i3 < NCHUNK)
        def _():
            issue_idx(i3, q3)

        @pl.when(i2 < NCHUNK)
        def _():
            @pl.when(i >= 2)
            def _():
                wait_scatter(b2)

            wait_idx(q2)
            issue_data(i2, b2, q2)

        pltpu.make_async_copy(node_hbm.at[idxs[q].at[0]], nbufs[b], gsems[b]).wait()
        pltpu.make_async_copy(edge_hbm.at[pl.ds(0, CH)], ebufs[b], lsems[b]).wait()

        nb = nbufs[b]
        eb = ebufs[b]

        def ebody(e, carry2):
            for j in range(8):
                sl = pl.ds(16 * j, 16)
                v = nb[e, sl] + eb[e, sl]
                eb[e, sl] = jnp.maximum(v, 0.01 * v)
            return carry2

        lax.fori_loop(0, CH, ebody, 0)
        pltpu.async_copy(eb, acc.at[idxs[q].at[1]], ssems[b], add=True)

    def outer(g, carry):
        for du in range(UNROLL):
            i = g * UNROLL + du
            step(i, du % NBUF, du % NIDX)
        return carry

    NFULL = (NCHUNK - 2) // UNROLL
    lax.fori_loop(0, NFULL, outer, 0)
    for t in range(NFULL * UNROLL, NCHUNK):
        step(jnp.int32(t), t % NBUF, t % NIDX)

    for j in range(NBUF):
        wait_scatter(j)

    plsc.subcore_barrier()

    pltpu.sync_copy(acc.at[pl.ds(row0, RPT)],
                    out_hbm.at[pl.ds(c * N_NODES + row0, RPT)])

    @pl.when(s == NS - 1)
    def _():
        pltpu.sync_copy(acc.at[pl.ds(row0 + RPT, 16)],
                        out_hbm.at[pl.ds(c * N_NODES + row0 + RPT, 16)])

  return _sc_scatter


_sc_scatter_a = _make_sc_scatter(0)
_sc_scatter_b = _make_sc_scatter(1)


def _final_add(pa, pb):
    def body(a_ref, b_ref, c_ref, d_ref, o_ref):
        o_ref[...] = (a_ref[...] + b_ref[...]) + (c_ref[...] + d_ref[...])

    return pl.pallas_call(
        body,
        out_shape=jax.ShapeDtypeStruct((N_NODES, D), jnp.float32),
    )(pa[:N_NODES], pa[N_NODES:], pb[:N_NODES], pb[N_NODES:])


def kernel(x, edge_index, edge_attr, Wn, bn, We, be, Wg, bg):
    src = edge_index[0].astype(jnp.int32)
    dst = edge_index[1].astype(jnp.int32)
    Wge = Wg[:DE, :]
    Wgn = Wg[DE:, :]
    node_part = _node_dense(x, Wn, bn.reshape(1, D), Wgn)
    ea_t = edge_attr.T
    be_c = be.reshape(DE, 1)
    bg_r = bg.reshape(1, D)
    edge_part_a = _edge_dense(ea_t, We, be_c, Wge, bg_r, 0)
    partials_a = _sc_scatter_a(node_part, edge_part_a, src, dst)
    edge_part_b = _edge_dense(ea_t, We, be_c, Wge, bg_r, 1)
    partials_b = _sc_scatter_b(node_part, edge_part_b, src, dst)
    return _final_add(partials_a, partials_b)


# speedup vs baseline: 1.0707x; 1.0707x over previous
import functools

import jax
import jax.numpy as jnp
from jax import lax
from jax.experimental import pallas as pl
from jax.experimental.pallas import tpu as pltpu
from jax.experimental.pallas import tpu_sc as plsc

N_NODES = 10000
N_EDGES = 320000
D = 128
DE = 16

NC = 2
NS = 16
NW = NC * NS
NSPLIT = 1
E_SPLIT = N_EDGES // NSPLIT
E_W = E_SPLIT // NW
CH = 40
NCHUNK = E_W // CH
RPT = 624


def _leaky(v):
    return jnp.maximum(v, 0.01 * v)


def _node_dense(x, Wn, bn, Wgn):
    def body(x_ref, wn_ref, bn_ref, wgn_ref, o_ref):
        h = jnp.dot(x_ref[...], wn_ref[...], preferred_element_type=jnp.float32)
        h = _leaky(h + bn_ref[...])
        o_ref[...] = jnp.dot(h, wgn_ref[...], preferred_element_type=jnp.float32)

    return pl.pallas_call(
        body,
        out_shape=jax.ShapeDtypeStruct((N_NODES, D), jnp.float32),
    )(x, Wn, bn, Wgn)


def _edge_dense(ea_t, We, be, Wge, bg, half):
    BLK = 16000
    assert E_SPLIT % BLK == 0
    grid = E_SPLIT // BLK
    boff = half * grid
    cdim = (((0,), (0,)), ((), ()))

    def body(a_ref, we_ref, be_ref, wge_ref, bg_ref, o_ref):
        ht = lax.dot_general(we_ref[...], a_ref[...], cdim,
                             preferred_element_type=jnp.float32)
        ht = _leaky(ht + be_ref[...])
        o_ref[...] = lax.dot_general(ht, wge_ref[...], cdim,
                                     preferred_element_type=jnp.float32) + bg_ref[...]

    return pl.pallas_call(
        body,
        grid=(grid,),
        in_specs=[
            pl.BlockSpec((DE, BLK), lambda i: (0, i + boff)),
            pl.BlockSpec((DE, DE), lambda i: (0, 0)),
            pl.BlockSpec((DE, 1), lambda i: (0, 0)),
            pl.BlockSpec((DE, D), lambda i: (0, 0)),
            pl.BlockSpec((1, D), lambda i: (0, 0)),
        ],
        out_specs=pl.BlockSpec((BLK, D), lambda i: (i, 0)),
        out_shape=jax.ShapeDtypeStruct((E_SPLIT, D), jnp.float32),
    )(ea_t, We, be, Wge, bg)


NBUF = 4
NIDX = 8
UNROLL = 8


def _make_sc_scatter(half):
  eoff = half * E_SPLIT

  @functools.partial(
      pl.kernel,
      out_type=jax.ShapeDtypeStruct((2 * N_NODES, D), jnp.float32),
      mesh=plsc.VectorSubcoreMesh(core_axis_name="c", subcore_axis_name="s"),
      scratch_types=[
          [pltpu.VMEM((2, CH), jnp.int32) for _ in range(NIDX)],
          [pltpu.VMEM((CH, D), jnp.float32) for _ in range(NBUF)],
          [pltpu.VMEM((CH, D), jnp.float32) for _ in range(NBUF)],
          [pltpu.SemaphoreType.DMA for _ in range(NIDX)],
          [pltpu.SemaphoreType.DMA for _ in range(NBUF)],
          [pltpu.SemaphoreType.DMA for _ in range(NBUF)],
          [pltpu.SemaphoreType.DMA for _ in range(NBUF)],
          pltpu.VMEM_SHARED((N_NODES, D), jnp.float32),
      ],
  )
  def _sc_scatter(node_hbm, edge_hbm, src_hbm, dst_hbm, out_hbm,
                  idxs, nbufs, ebufs, isems, gsems, lsems, ssems, acc):
    c = lax.axis_index("c")
    s = lax.axis_index("s")
    wid = c * NS + s
    ebase = wid * E_W
    gbase = eoff + ebase

    z = ebufs[0]

    @plsc.parallel_loop(0, CH, unroll=4)
    def _(e):
        for j in range(8):
            z[e, pl.ds(16 * j, 16)] = jnp.zeros((16,), jnp.float32)

    row0 = s * RPT
    for k in range(15):
        pltpu.sync_copy(z, acc.at[pl.ds(row0 + k * CH, CH)])
    pltpu.sync_copy(z.at[pl.ds(0, 24)], acc.at[pl.ds(row0 + 600, 24)])

    @pl.when(s == NS - 1)
    def _():
        pltpu.sync_copy(z.at[pl.ds(0, 16)], acc.at[pl.ds(row0 + RPT, 16)])

    plsc.subcore_barrier()

    def issue_idx(i, q):
        pltpu.async_copy(src_hbm.at[pl.ds(gbase + i * CH, CH)], idxs[q].at[0], isems[q])
        pltpu.async_copy(dst_hbm.at[pl.ds(gbase + i * CH, CH)], idxs[q].at[1], isems[q])

    def wait_idx(q):
        pltpu.make_async_copy(src_hbm.at[pl.ds(0, CH)], idxs[q].at[0], isems[q]).wait()
        pltpu.make_async_copy(src_hbm.at[pl.ds(0, CH)], idxs[q].at[1], isems[q]).wait()

    def issue_data(i, b, q):
        pltpu.async_copy(node_hbm.at[idxs[q].at[0]], nbufs[b], gsems[b])
        pltpu.async_copy(edge_hbm.at[pl.ds(ebase + i * CH, CH)], ebufs[b], lsems[b])

    def wait_scatter(b):
        pltpu.make_async_copy(ebufs[b], acc.at[idxs[0].at[1]], ssems[b]).wait()

    for j in range(3):
        issue_idx(j, j)
    for j in range(2):
        wait_idx(j)
        issue_data(j, j, j)

    def step(i, b, q):
        b2 = (b + 2) % NBUF
        q2 = (q + 2) % NIDX
        q3 = (q + 3) % NIDX
        i2 = i + 2
        i3 = i + 3

        @pl.when(i3 < NCHUNK)
        def _():
            issue_idx(i3, q3)

        @pl.when(i2 < NCHUNK)
        def _():
            @pl.when(i >= 2)
            def _():
                wait_scatter(b2)

            wait_idx(q2)
            issue_data(i2, b2, q2)

        pltpu.make_async_copy(node_hbm.at[idxs[q].at[0]], nbufs[b], gsems[b]).wait()
        pltpu.make_async_copy(edge_hbm.at[pl.ds(0, CH)], ebufs[b], lsems[b]).wait()

        nb = nbufs[b]
        eb = ebufs[b]

        def ebody(e, carry2):
            for j in range(8):
                sl = pl.ds(16 * j, 16)
                v = nb[e, sl] + eb[e, sl]
                eb[e, sl] = jnp.maximum(v, 0.01 * v)
            return carry2

        lax.fori_loop(0, CH, ebody, 0)
        pltpu.async_copy(eb, acc.at[idxs[q].at[1]], ssems[b], add=True)

    def outer(g, carry):
        for du in range(UNROLL):
            i = g * UNROLL + du
            step(i, du % NBUF, du % NIDX)
        return carry

    NFULL = (NCHUNK - 2) // UNROLL
    lax.fori_loop(0, NFULL, outer, 0)
    for t in range(NFULL * UNROLL, NCHUNK):
        step(jnp.int32(t), t % NBUF, t % NIDX)

    for j in range(NBUF):
        wait_scatter(j)

    plsc.subcore_barrier()

    pltpu.sync_copy(acc.at[pl.ds(row0, RPT)],
                    out_hbm.at[pl.ds(c * N_NODES + row0, RPT)])

    @pl.when(s == NS - 1)
    def _():
        pltpu.sync_copy(acc.at[pl.ds(row0 + RPT, 16)],
                        out_hbm.at[pl.ds(c * N_NODES + row0 + RPT, 16)])

  return _sc_scatter


_sc_scatters = [_make_sc_scatter(h) for h in range(NSPLIT)]


def _final_add(parts):
    def body(*refs):
        o_ref = refs[-1]
        acc = refs[0][...] + refs[1][...]
        for r in refs[2:-1]:
            acc = acc + r[...]
        o_ref[...] = acc

    halves = []
    for p in parts:
        halves.append(p[:N_NODES])
        halves.append(p[N_NODES:])
    return pl.pallas_call(
        body,
        out_shape=jax.ShapeDtypeStruct((N_NODES, D), jnp.float32),
    )(*halves)


def kernel(x, edge_index, edge_attr, Wn, bn, We, be, Wg, bg):
    src = edge_index[0].astype(jnp.int32)
    dst = edge_index[1].astype(jnp.int32)
    Wge = Wg[:DE, :]
    Wgn = Wg[DE:, :]
    node_part = _node_dense(x, Wn, bn.reshape(1, D), Wgn)
    ea_t = edge_attr.T
    be_c = be.reshape(DE, 1)
    bg_r = bg.reshape(1, D)
    parts = []
    for h in range(NSPLIT):
        edge_part_h = _edge_dense(ea_t, We, be_c, Wge, bg_r, h)
        parts.append(_sc_scatters[h](node_part, edge_part_h, src, dst))
    return _final_add(parts)
